# baseline (speedup 1.0000x reference)
import functools

import jax
import jax.numpy as jnp
from jax import lax
from jax.experimental import pallas as pl
from jax.experimental.pallas import tpu as pltpu
from jax.experimental.pallas import tpu_sc as plsc

NC = 2
NS = 16
NW = NC * NS
LANES = 16

BATCH = 16384
HALF = BATCH // 2
PER_W = HALF // NW
ROWS = 2 * PER_W
DIM = 64
GCHUNK = 128
NG = ROWS // GCHUNK
MARGIN_F = 1.0


def _rsqrt_nr(s):
    i = plsc.bitcast(s, jnp.int32)
    i = jnp.full((LANES,), 0x5F3759DF, jnp.int32) - lax.shift_right_logical(
        i, jnp.full((LANES,), 1, jnp.int32))
    y = plsc.bitcast(i, jnp.float32)
    half_s = 0.5 * s
    for _ in range(3):
        y = y * (1.5 - half_s * y * y)
    return y


def _body(h_idx_hbm, t_idx_hbm, r_idx_hbm, ent_hbm, rel_hbm,
          pos_out, neg_out, loss_out,
          h_idx_v, t_idx_v, r_idx_v, h_rows, t_rows, r_rows,
          scores_v, loss_v, sem):
    w = lax.axis_index("s") * NC + lax.axis_index("c")

    pltpu.sync_copy(h_idx_hbm.at[w], h_idx_v)
    pltpu.sync_copy(t_idx_hbm.at[w], t_idx_v)
    pltpu.sync_copy(r_idx_hbm.at[w], r_idx_v)

    copies = []
    for k in range(NG):
        dst = pl.ds(k * GCHUNK, GCHUNK)
        copies.append(pltpu.async_copy(ent_hbm.at[h_idx_v.at[k]],
                                       h_rows.at[dst], sem))
        copies.append(pltpu.async_copy(ent_hbm.at[t_idx_v.at[k]],
                                       t_rows.at[dst], sem))
        copies.append(pltpu.async_copy(rel_hbm.at[r_idx_v.at[k]],
                                       r_rows.at[dst], sem))
    for c in copies:
        c.wait()

    def group(g, carry):
        rows = g * LANES + lax.iota(jnp.int32, LANES)
        sh = jnp.zeros((LANES,), jnp.float32)
        st = jnp.zeros((LANES,), jnp.float32)
        for j in range(DIM):
            cj = jnp.full((LANES,), j, jnp.int32)
            hv = plsc.load_gather(h_rows, [rows, cj])
            tv = plsc.load_gather(t_rows, [rows, cj])
            sh = sh + hv * hv
            st = st + tv * tv
        one = jnp.full((LANES,), 1.0, jnp.float32)
        sc_h = jnp.where(sh > one, _rsqrt_nr(sh), one)
        sc_t = jnp.where(st > one, _rsqrt_nr(st), one)
        acc = jnp.zeros((LANES,), jnp.float32)
        for j in range(DIM):
            cj = jnp.full((LANES,), j, jnp.int32)
            hv = plsc.load_gather(h_rows, [rows, cj])
            tv = plsc.load_gather(t_rows, [rows, cj])
            rv = plsc.load_gather(r_rows, [rows, cj])
            acc = acc + jnp.abs(hv * sc_h + rv - tv * sc_t)
        scores_v[pl.ds(g * LANES, LANES)] = acc
        return carry

    lax.fori_loop(0, ROWS // LANES, group, 0)

    def pair(m, lacc):
        pv = scores_v[pl.ds(m * LANES, LANES)]
        nv = scores_v[pl.ds(PER_W + m * LANES, LANES)]
        return lacc + jnp.maximum(pv - nv + MARGIN_F, 0.0)

    lacc = lax.fori_loop(0, PER_W // LANES, pair,
                         jnp.zeros((LANES,), jnp.float32))
    loss_v[...] = lacc

    pltpu.sync_copy(scores_v.at[pl.ds(0, PER_W)],
                    pos_out.at[pl.ds(w * PER_W, PER_W)])
    pltpu.sync_copy(scores_v.at[pl.ds(PER_W, PER_W)],
                    neg_out.at[pl.ds(w * PER_W, PER_W)])
    pltpu.sync_copy(loss_v, loss_out.at[w])


_sc_call = pl.kernel(
    _body,
    out_type=(
        jax.ShapeDtypeStruct((HALF,), jnp.float32),
        jax.ShapeDtypeStruct((HALF,), jnp.float32),
        jax.ShapeDtypeStruct((NW, LANES), jnp.float32),
    ),
    mesh=plsc.VectorSubcoreMesh(core_axis_name="c", subcore_axis_name="s",
                                num_cores=NC, num_subcores=NS),
    compiler_params=pltpu.CompilerParams(
        needs_layout_passes=False, use_tc_tiling_on_sc=False),
    scratch_types=[
        pltpu.VMEM((NG, GCHUNK), jnp.int32),
        pltpu.VMEM((NG, GCHUNK), jnp.int32),
        pltpu.VMEM((NG, GCHUNK), jnp.int32),
        pltpu.VMEM((ROWS, DIM), jnp.float32),
        pltpu.VMEM((ROWS, DIM), jnp.float32),
        pltpu.VMEM((ROWS, DIM), jnp.float32),
        pltpu.VMEM((ROWS,), jnp.float32),
        pltpu.VMEM((LANES,), jnp.float32),
        pltpu.SemaphoreType.DMA,
    ],
)


def _split_idx(x):
    pos = x[:HALF].reshape(NW, NG // 2, GCHUNK)
    neg = x[HALF:].reshape(NW, NG // 2, GCHUNK)
    return jnp.concatenate([pos, neg], axis=1)


@jax.jit
def kernel(batch_h, batch_t, batch_r, batch_y, ent_emb, rel_emb):
    del batch_y
    pos, neg, loss_parts = _sc_call(
        _split_idx(batch_h), _split_idx(batch_t), _split_idx(batch_r),
        ent_emb, rel_emb)
    return (jnp.sum(loss_parts), pos, neg)


# speedup vs baseline: 2.2109x; 2.2109x over previous
import jax
import jax.numpy as jnp
from jax import lax
from jax.experimental import pallas as pl
from jax.experimental.pallas import tpu as pltpu
from jax.experimental.pallas import tpu_sc as plsc

NC = 2
NS = 16
NW = NC * NS
LANES = 16

BATCH = 16384
HALF = BATCH // 2
PER_W = HALF // NW
DIM = 64
TOTAL_ENT = 1000000
RANGE = TOTAL_ENT // NW
NTC = 246
NSUB = 16
SUBCAP = 256
MYCAP = 2048
GROWS = 2 * BATCH
GDUMP = 32
RB = 8
MARGIN_F = 1.0


def _iota16():
    return lax.iota(jnp.int32, LANES)


def _stage1_body(h_ids_hbm, t_ids_hbm, ent_t, g_out,
                 hids_v, tids_v, my_ids, my_slots, subids, subslots,
                 wk_ids, wk_slots, bufA, bufB, rb, oi, smem,
                 semA, semB, semS):
    w = lax.axis_index("s") * NC + lax.axis_index("c")
    lo = w * RANGE
    hi = lo + RANGE
    base_tc = lax.div(lo, 128)
    end_tc = lax.div(hi - 1, 128)
    ntc = end_tc - base_tc + 1
    dump_row = GROWS + w % GDUMP

    pltpu.sync_copy(h_ids_hbm, hids_v)
    pltpu.sync_copy(t_ids_hbm, tids_v)

    def scan(src_v, slot_off):
        def body(v, cnt):
            ids = src_v[pl.ds(v * LANES, LANES)]
            m = (ids >= lo) & (ids < hi)
            plsc.store_compressed(my_ids.at[pl.ds(cnt, LANES)], ids, mask=m)
            sl = slot_off + v * LANES + _iota16()
            plsc.store_compressed(my_slots.at[pl.ds(cnt, LANES)], sl, mask=m)
            return cnt + plsc.all_reduce_population_count(m)[0]
        return body

    mcnt = lax.fori_loop(0, BATCH // LANES, scan(hids_v, 0), 0)
    mcnt = lax.fori_loop(0, BATCH // LANES, scan(tids_v, BATCH), mcnt)

    def bucket(v, cnts):
        ids = my_ids[pl.ds(v * LANES, LANES)]
        sls = my_slots[pl.ds(v * LANES, LANES)]
        valid = (v * LANES + _iota16()) < mcnt
        sub = lax.shift_right_logical(
            lax.shift_right_logical(ids, jnp.full((LANES,), 7, jnp.int32))
            - base_tc, jnp.full((LANES,), 4, jnp.int32))
        new = []
        for j in range(NSUB):
            mj = valid & (sub == j)
            cj = cnts[j]
            plsc.store_compressed(
                subids.at[pl.ds(j * SUBCAP + cj, LANES)], ids, mask=mj)
            plsc.store_compressed(
                subslots.at[pl.ds(j * SUBCAP + cj, LANES)], sls, mask=mj)
            new.append(cj + plsc.all_reduce_population_count(mj)[0])
        return tuple(new)

    subcnt = lax.fori_loop(0, (mcnt + LANES - 1) // LANES, bucket,
                           (0,) * NSUB)
    for j in range(NSUB):
        smem[j] = subcnt[j]

    def fire(c, buf, sem):
        off = pl.multiple_of((base_tc + c) * 128, 128)
        pltpu.async_copy(ent_t.at[:, pl.ds(off, 128)], buf, sem)

    def drain_fetch(buf, sem):
        pltpu.make_async_copy(ent_t.at[:, pl.ds(0, 128)], buf, sem).wait()

    def drain_scatter():
        pltpu.make_async_copy(rb.at[0], g_out.at[oi.at[0]], semS).wait()

    def init_oi(b):
        plsc.store_scatter(oi.at[b], [_iota16()],
                           jnp.full((LANES,), dump_row, jnp.int32))

    for b in range(RB):
        init_oi(b)

    def process(buf, c, state):
        tc = base_tc + c
        sub = lax.div(c, NSUB)
        scnt = smem[sub]

        def svreg(v, st):
            sids = subids[pl.ds(sub * SUBCAP + v * LANES, LANES)]
            ssl = subslots[pl.ds(sub * SUBCAP + v * LANES, LANES)]
            valid = (v * LANES + _iota16()) < scnt
            m = valid & (lax.shift_right_logical(
                sids, jnp.full((LANES,), 7, jnp.int32)) == tc)
            n = plsc.all_reduce_population_count(m)[0]
            plsc.store_compressed(wk_ids.at[pl.ds(0, LANES)], sids, mask=m)
            plsc.store_compressed(wk_slots.at[pl.ds(0, LANES)], ssl, mask=m)

            def ext(e, st2):
                cur, bb, fired, drained = st2
                idv = wk_ids[pl.ds(e, LANES)]
                slv = wk_slots[pl.ds(e, LANES)]
                col = jnp.full((LANES,), idv[0] & 127, jnp.int32)
                rows = _iota16()
                for k in range(DIM // LANES):
                    vk = plsc.load_gather(buf, [rows + k * LANES, col])
                    rb[bb, cur, pl.ds(k * LANES, LANES)] = vk
                plsc.store_scatter(
                    oi.at[bb], [jnp.full((LANES,), cur, jnp.int32)],
                    jnp.full((LANES,), slv[0], jnp.int32),
                    mask=(_iota16() == 0))
                full = cur == LANES - 1

                @pl.when(full)
                def _():
                    pltpu.async_copy(rb.at[bb], g_out.at[oi.at[bb]], semS)

                @pl.when(full & (fired >= RB - 1))
                def _():
                    drain_scatter()

                bb2 = jnp.where(full, lax.rem(bb + 1, RB), bb)

                @pl.when(full)
                def _():
                    init_oi(bb2)

                return (jnp.where(full, 0, cur + 1),
                        bb2,
                        jnp.where(full, fired + 1, fired),
                        jnp.where(full & (fired >= RB - 1),
                                  drained + 1, drained))

            return lax.fori_loop(0, n, ext, st)

        nv = lax.div(scnt + LANES - 1, LANES)
        return lax.fori_loop(0, nv, svreg, state)

    fire(0, bufA, semA)
    state = (0, 0, 0, 0)

    def pair(i, state):
        c0 = 2 * i
        c1 = c0 + 1

        @pl.when(c1 < ntc)
        def _():
            fire(c1, bufB, semB)

        drain_fetch(bufA, semA)
        state = process(bufA, c0, state)

        @pl.when(c0 + 2 < ntc)
        def _():
            fire(c0 + 2, bufA, semA)

        def do_b(st):
            drain_fetch(bufB, semB)
            return process(bufB, c1, st)

        return lax.cond(c1 < ntc, do_b, lambda st: st, state)

    state = lax.fori_loop(0, NTC // 2, pair, state)
    cur, bb, fired, drained = state

    @pl.when(cur > 0)
    def _():
        pltpu.async_copy(rb.at[bb], g_out.at[oi.at[bb]], semS)

    fired = jnp.where(cur > 0, fired + 1, fired)

    def fin(i, d):
        drain_scatter()
        return d + 1

    lax.fori_loop(0, fired - drained, fin, drained)


_stage1 = pl.kernel(
    _stage1_body,
    out_type=jax.ShapeDtypeStruct((GROWS + GDUMP, 128), jnp.float32),
    mesh=plsc.VectorSubcoreMesh(core_axis_name="c", subcore_axis_name="s",
                                num_cores=NC, num_subcores=NS),
    compiler_params=pltpu.CompilerParams(
        needs_layout_passes=False, use_tc_tiling_on_sc=True),
    scratch_types=[
        pltpu.VMEM((BATCH,), jnp.int32),
        pltpu.VMEM((BATCH,), jnp.int32),
        pltpu.VMEM((MYCAP,), jnp.int32),
        pltpu.VMEM((MYCAP,), jnp.int32),
        pltpu.VMEM((NSUB * SUBCAP,), jnp.int32),
        pltpu.VMEM((NSUB * SUBCAP,), jnp.int32),
        pltpu.VMEM((32,), jnp.int32),
        pltpu.VMEM((32,), jnp.int32),
        pltpu.VMEM((DIM, 128), jnp.float32),
        pltpu.VMEM((DIM, 128), jnp.float32),
        pltpu.VMEM((RB, LANES, 128), jnp.float32),
        pltpu.VMEM((RB, LANES), jnp.int32),
        pltpu.SMEM((NSUB,), jnp.int32),
        pltpu.SemaphoreType.DMA,
        pltpu.SemaphoreType.DMA,
        pltpu.SemaphoreType.DMA,
    ],
)


def _rsqrt_nr(s):
    i = plsc.bitcast(s, jnp.int32)
    i = jnp.full((LANES,), 0x5F3759DF, jnp.int32) - lax.shift_right_logical(
        i, jnp.full((LANES,), 1, jnp.int32))
    y = plsc.bitcast(i, jnp.float32)
    half_s = 0.5 * s
    for _ in range(3):
        y = y * (1.5 - half_s * y * y)
    return y


def _stage2_body(g_in, r_idx_hbm, rel_pad, pos_out, neg_out, loss_out,
                 r_idx_v, hbuf, tbuf, rbuf, scores_v, loss_v, sem):
    w = lax.axis_index("s") * NC + lax.axis_index("c")
    pltpu.sync_copy(r_idx_hbm.at[w], r_idx_v)

    for half in range(2):
        slot0 = half * HALF + w * PER_W
        cps = [
            pltpu.async_copy(g_in.at[pl.ds(slot0, PER_W)], hbuf, sem),
            pltpu.async_copy(g_in.at[pl.ds(BATCH + slot0, PER_W)], tbuf, sem),
            pltpu.async_copy(rel_pad.at[r_idx_v.at[2 * half]],
                             rbuf.at[pl.ds(0, 128)], sem),
            pltpu.async_copy(rel_pad.at[r_idx_v.at[2 * half + 1]],
                             rbuf.at[pl.ds(128, 128)], sem),
        ]
        for c in cps:
            c.wait()

        def group(g, carry):
            rows = g * LANES + _iota16()
            sh = jnp.zeros((LANES,), jnp.float32)
            st = jnp.zeros((LANES,), jnp.float32)
            for j in range(DIM):
                cj = jnp.full((LANES,), j, jnp.int32)
                hv = plsc.load_gather(hbuf, [rows, cj])
                tv = plsc.load_gather(tbuf, [rows, cj])
                sh = sh + hv * hv
                st = st + tv * tv
            one = jnp.full((LANES,), 1.0, jnp.float32)
            sc_h = jnp.where(sh > one, _rsqrt_nr(sh), one)
            sc_t = jnp.where(st > one, _rsqrt_nr(st), one)
            acc = jnp.zeros((LANES,), jnp.float32)
            for j in range(DIM):
                cj = jnp.full((LANES,), j, jnp.int32)
                hv = plsc.load_gather(hbuf, [rows, cj])
                tv = plsc.load_gather(tbuf, [rows, cj])
                rv = plsc.load_gather(rbuf, [rows, cj])
                acc = acc + jnp.abs(hv * sc_h + rv - tv * sc_t)
            scores_v[pl.ds(half * PER_W + g * LANES, LANES)] = acc
            return carry

        lax.fori_loop(0, PER_W // LANES, group, 0)

    def pair(m, lacc):
        pv = scores_v[pl.ds(m * LANES, LANES)]
        nv = scores_v[pl.ds(PER_W + m * LANES, LANES)]
        return lacc + jnp.maximum(pv - nv + MARGIN_F, 0.0)

    lacc = lax.fori_loop(0, PER_W // LANES, pair,
                         jnp.zeros((LANES,), jnp.float32))
    loss_v[...] = lacc

    pltpu.sync_copy(scores_v.at[pl.ds(0, PER_W)],
                    pos_out.at[pl.ds(w * PER_W, PER_W)])
    pltpu.sync_copy(scores_v.at[pl.ds(PER_W, PER_W)],
                    neg_out.at[pl.ds(w * PER_W, PER_W)])
    pltpu.sync_copy(loss_v, loss_out.at[w])


_stage2 = pl.kernel(
    _stage2_body,
    out_type=(
        jax.ShapeDtypeStruct((HALF,), jnp.float32),
        jax.ShapeDtypeStruct((HALF,), jnp.float32),
        jax.ShapeDtypeStruct((NW, LANES), jnp.float32),
    ),
    mesh=plsc.VectorSubcoreMesh(core_axis_name="c", subcore_axis_name="s",
                                num_cores=NC, num_subcores=NS),
    compiler_params=pltpu.CompilerParams(
        needs_layout_passes=False, use_tc_tiling_on_sc=True),
    scratch_types=[
        pltpu.VMEM((4, 128), jnp.int32),
        pltpu.VMEM((PER_W, 128), jnp.float32),
        pltpu.VMEM((PER_W, 128), jnp.float32),
        pltpu.VMEM((PER_W, 128), jnp.float32),
        pltpu.VMEM((2 * PER_W,), jnp.float32),
        pltpu.VMEM((LANES,), jnp.float32),
        pltpu.SemaphoreType.DMA,
    ],
)


def _split_idx(x):
    pos = x[:HALF].reshape(NW, 2, 128)
    neg = x[HALF:].reshape(NW, 2, 128)
    return jnp.concatenate([pos, neg], axis=1)


@jax.jit
def kernel(batch_h, batch_t, batch_r, batch_y, ent_emb, rel_emb):
    del batch_y
    g = _stage1(batch_h, batch_t, ent_emb.T)
    rel_pad = jnp.pad(rel_emb, ((0, 0), (0, 128 - DIM)))
    pos, neg, loss_parts = _stage2(g, _split_idx(batch_r), rel_pad)
    return (jnp.sum(loss_parts), pos, neg)


# speedup vs baseline: 2.5421x; 1.1498x over previous
import jax
import jax.numpy as jnp
from jax import lax
from jax.experimental import pallas as pl
from jax.experimental.pallas import tpu as pltpu
from jax.experimental.pallas import tpu_sc as plsc

NC = 2
NS = 16
NW = NC * NS
LANES = 16

BATCH = 16384
HALF = BATCH // 2
PER_W = HALF // NW
DIM = 64
TOTAL_ENT = 1000000
RANGE = TOTAL_ENT // NW
NTC = 246
NSUB = 16
SUBCAP = 256
MYCAP = 2048
GROWS = 2 * BATCH
GDUMP = 32
RB = 8
MARGIN_F = 1.0


def _iota16():
    return lax.iota(jnp.int32, LANES)


def _stage1_body(h_ids_hbm, t_ids_hbm, ent_t, g_out,
                 hids_v, tids_v, my_ids, my_slots, subids, subslots,
                 wk_ids, wk_slots, bufA, bufB, rb, oi, smem,
                 semA, semB, semS):
    w = lax.axis_index("s") * NC + lax.axis_index("c")
    lo = w * RANGE
    hi = lo + RANGE
    base_tc = lax.div(lo, 128)
    end_tc = lax.div(hi - 1, 128)
    ntc = end_tc - base_tc + 1
    dump_row = GROWS + w % GDUMP

    pltpu.sync_copy(h_ids_hbm, hids_v)
    pltpu.sync_copy(t_ids_hbm, tids_v)

    def scan(src_v, slot_off):
        def body(v, cnt):
            ids = src_v[pl.ds(v * LANES, LANES)]
            m = (ids >= lo) & (ids < hi)
            plsc.store_compressed(my_ids.at[pl.ds(cnt, LANES)], ids, mask=m)
            sl = slot_off + v * LANES + _iota16()
            plsc.store_compressed(my_slots.at[pl.ds(cnt, LANES)], sl, mask=m)
            return cnt + plsc.all_reduce_population_count(m)[0]
        return body

    mcnt = lax.fori_loop(0, BATCH // LANES, scan(hids_v, 0), 0)
    mcnt = lax.fori_loop(0, BATCH // LANES, scan(tids_v, BATCH), mcnt)

    def bucket(v, cnts):
        ids = my_ids[pl.ds(v * LANES, LANES)]
        sls = my_slots[pl.ds(v * LANES, LANES)]
        valid = (v * LANES + _iota16()) < mcnt
        sub = lax.shift_right_logical(
            lax.shift_right_logical(ids, jnp.full((LANES,), 7, jnp.int32))
            - base_tc, jnp.full((LANES,), 4, jnp.int32))
        new = []
        for j in range(NSUB):
            mj = valid & (sub == j)
            cj = cnts[j]
            plsc.store_compressed(
                subids.at[pl.ds(j * SUBCAP + cj, LANES)], ids, mask=mj)
            plsc.store_compressed(
                subslots.at[pl.ds(j * SUBCAP + cj, LANES)], sls, mask=mj)
            new.append(cj + plsc.all_reduce_population_count(mj)[0])
        return tuple(new)

    subcnt = lax.fori_loop(0, (mcnt + LANES - 1) // LANES, bucket,
                           (0,) * NSUB)
    for j in range(NSUB):
        smem[j] = subcnt[j]

    def fire(f, buf, sem):
        tc0 = jnp.minimum(base_tc + 2 * f, (TOTAL_ENT // 128) - 1)
        off = pl.multiple_of(tc0 * 128, 128)
        pltpu.async_copy(ent_t.at[:, pl.ds(off, 256)], buf, sem)

    def drain_fetch(buf, sem):
        pltpu.make_async_copy(ent_t.at[:, pl.ds(0, 256)], buf, sem).wait()

    def drain_scatter():
        pltpu.make_async_copy(rb.at[0], g_out.at[oi.at[0]], semS).wait()

    def init_oi(b):
        plsc.store_scatter(oi.at[b], [_iota16()],
                           jnp.full((LANES,), dump_row, jnp.int32))

    for b in range(RB):
        init_oi(b)

    def process(buf, c, col_base, state):
        tc = base_tc + c
        sub = lax.div(c, NSUB)
        scnt = smem[sub]

        def svreg(v, st):
            sids = subids[pl.ds(sub * SUBCAP + v * LANES, LANES)]
            ssl = subslots[pl.ds(sub * SUBCAP + v * LANES, LANES)]
            valid = (v * LANES + _iota16()) < scnt
            m = valid & (lax.shift_right_logical(
                sids, jnp.full((LANES,), 7, jnp.int32)) == tc)
            n = plsc.all_reduce_population_count(m)[0]
            plsc.store_compressed(wk_ids.at[pl.ds(0, LANES)], sids, mask=m)
            plsc.store_compressed(wk_slots.at[pl.ds(0, LANES)], ssl, mask=m)

            def ext(e, st2):
                cur, bb, fired, drained = st2
                idv = wk_ids[pl.ds(e, LANES)]
                slv = wk_slots[pl.ds(e, LANES)]
                col = jnp.full((LANES,), (idv[0] & 127) + col_base,
                               jnp.int32)
                rows = _iota16()
                for k in range(DIM // LANES):
                    vk = plsc.load_gather(buf, [rows + k * LANES, col])
                    rb[bb, cur, pl.ds(k * LANES, LANES)] = vk
                plsc.store_scatter(
                    oi.at[bb], [jnp.full((LANES,), cur, jnp.int32)],
                    jnp.full((LANES,), slv[0], jnp.int32),
                    mask=(_iota16() == 0))
                full = cur == LANES - 1

                @pl.when(full)
                def _():
                    pltpu.async_copy(rb.at[bb], g_out.at[oi.at[bb]], semS)

                @pl.when(full & (fired >= RB - 1))
                def _():
                    drain_scatter()

                bb2 = jnp.where(full, lax.rem(bb + 1, RB), bb)

                @pl.when(full)
                def _():
                    init_oi(bb2)

                return (jnp.where(full, 0, cur + 1),
                        bb2,
                        jnp.where(full, fired + 1, fired),
                        jnp.where(full & (fired >= RB - 1),
                                  drained + 1, drained))

            return lax.fori_loop(0, n, ext, st)

        nv = lax.div(scnt + LANES - 1, LANES)
        return lax.fori_loop(0, nv, svreg, state)

    fire(0, bufA, semA)
    state = (0, 0, 0, 0)

    def proc2(buf, f, st):
        tc0f = jnp.minimum(base_tc + 2 * f, (TOTAL_ENT // 128) - 1)
        cb0 = (base_tc + 2 * f - tc0f) * 128
        st = process(buf, 2 * f, cb0, st)

        def second(s2):
            return process(buf, 2 * f + 1, cb0 + 128, s2)

        return lax.cond(2 * f + 1 < ntc, second, lambda s2: s2, st)

    def pair(i, state):
        f0 = 2 * i
        f1 = f0 + 1

        @pl.when(2 * f1 < ntc)
        def _():
            fire(f1, bufB, semB)

        drain_fetch(bufA, semA)
        state = proc2(bufA, f0, state)

        @pl.when(2 * (f0 + 2) < ntc)
        def _():
            fire(f0 + 2, bufA, semA)

        def do_b(st):
            drain_fetch(bufB, semB)
            return proc2(bufB, f1, st)

        return lax.cond(2 * f1 < ntc, do_b, lambda st: st, state)

    state = lax.fori_loop(0, (NTC // 2 + 1) // 2, pair, state)
    cur, bb, fired, drained = state

    @pl.when(cur > 0)
    def _():
        pltpu.async_copy(rb.at[bb], g_out.at[oi.at[bb]], semS)

    fired = jnp.where(cur > 0, fired + 1, fired)

    def fin(i, d):
        drain_scatter()
        return d + 1

    lax.fori_loop(0, fired - drained, fin, drained)


_stage1 = pl.kernel(
    _stage1_body,
    out_type=jax.ShapeDtypeStruct((GROWS + GDUMP, 128), jnp.float32),
    mesh=plsc.VectorSubcoreMesh(core_axis_name="c", subcore_axis_name="s",
                                num_cores=NC, num_subcores=NS),
    compiler_params=pltpu.CompilerParams(
        needs_layout_passes=False, use_tc_tiling_on_sc=True),
    scratch_types=[
        pltpu.VMEM((BATCH,), jnp.int32),
        pltpu.VMEM((BATCH,), jnp.int32),
        pltpu.VMEM((MYCAP,), jnp.int32),
        pltpu.VMEM((MYCAP,), jnp.int32),
        pltpu.VMEM((NSUB * SUBCAP,), jnp.int32),
        pltpu.VMEM((NSUB * SUBCAP,), jnp.int32),
        pltpu.VMEM((32,), jnp.int32),
        pltpu.VMEM((32,), jnp.int32),
        pltpu.VMEM((DIM, 256), jnp.float32),
        pltpu.VMEM((DIM, 256), jnp.float32),
        pltpu.VMEM((RB, LANES, 128), jnp.float32),
        pltpu.VMEM((RB, LANES), jnp.int32),
        pltpu.SMEM((NSUB,), jnp.int32),
        pltpu.SemaphoreType.DMA,
        pltpu.SemaphoreType.DMA,
        pltpu.SemaphoreType.DMA,
    ],
)


def _rsqrt_nr(s):
    i = plsc.bitcast(s, jnp.int32)
    i = jnp.full((LANES,), 0x5F3759DF, jnp.int32) - lax.shift_right_logical(
        i, jnp.full((LANES,), 1, jnp.int32))
    y = plsc.bitcast(i, jnp.float32)
    half_s = 0.5 * s
    for _ in range(3):
        y = y * (1.5 - half_s * y * y)
    return y


def _stage2_body(g_in, r_idx_hbm, rel_pad, pos_out, neg_out, loss_out,
                 r_idx_v, hbuf, tbuf, rbuf, scores_v, loss_v, sem):
    w = lax.axis_index("s") * NC + lax.axis_index("c")
    pltpu.sync_copy(r_idx_hbm.at[w], r_idx_v)

    for half in range(2):
        slot0 = half * HALF + w * PER_W
        cps = [
            pltpu.async_copy(g_in.at[pl.ds(slot0, PER_W)], hbuf, sem),
            pltpu.async_copy(g_in.at[pl.ds(BATCH + slot0, PER_W)], tbuf, sem),
            pltpu.async_copy(rel_pad.at[r_idx_v.at[2 * half]],
                             rbuf.at[pl.ds(0, 128)], sem),
            pltpu.async_copy(rel_pad.at[r_idx_v.at[2 * half + 1]],
                             rbuf.at[pl.ds(128, 128)], sem),
        ]
        for c in cps:
            c.wait()

        def group(g, carry):
            rows = g * LANES + _iota16()
            sh = jnp.zeros((LANES,), jnp.float32)
            st = jnp.zeros((LANES,), jnp.float32)
            for j in range(DIM):
                cj = jnp.full((LANES,), j, jnp.int32)
                hv = plsc.load_gather(hbuf, [rows, cj])
                tv = plsc.load_gather(tbuf, [rows, cj])
                sh = sh + hv * hv
                st = st + tv * tv
            one = jnp.full((LANES,), 1.0, jnp.float32)
            sc_h = jnp.where(sh > one, _rsqrt_nr(sh), one)
            sc_t = jnp.where(st > one, _rsqrt_nr(st), one)
            acc = jnp.zeros((LANES,), jnp.float32)
            for j in range(DIM):
                cj = jnp.full((LANES,), j, jnp.int32)
                hv = plsc.load_gather(hbuf, [rows, cj])
                tv = plsc.load_gather(tbuf, [rows, cj])
                rv = plsc.load_gather(rbuf, [rows, cj])
                acc = acc + jnp.abs(hv * sc_h + rv - tv * sc_t)
            scores_v[pl.ds(half * PER_W + g * LANES, LANES)] = acc
            return carry

        lax.fori_loop(0, PER_W // LANES, group, 0)

    def pair(m, lacc):
        pv = scores_v[pl.ds(m * LANES, LANES)]
        nv = scores_v[pl.ds(PER_W + m * LANES, LANES)]
        return lacc + jnp.maximum(pv - nv + MARGIN_F, 0.0)

    lacc = lax.fori_loop(0, PER_W // LANES, pair,
                         jnp.zeros((LANES,), jnp.float32))
    loss_v[...] = lacc

    pltpu.sync_copy(scores_v.at[pl.ds(0, PER_W)],
                    pos_out.at[pl.ds(w * PER_W, PER_W)])
    pltpu.sync_copy(scores_v.at[pl.ds(PER_W, PER_W)],
                    neg_out.at[pl.ds(w * PER_W, PER_W)])
    pltpu.sync_copy(loss_v, loss_out.at[w])


_stage2 = pl.kernel(
    _stage2_body,
    out_type=(
        jax.ShapeDtypeStruct((HALF,), jnp.float32),
        jax.ShapeDtypeStruct((HALF,), jnp.float32),
        jax.ShapeDtypeStruct((NW, LANES), jnp.float32),
    ),
    mesh=plsc.VectorSubcoreMesh(core_axis_name="c", subcore_axis_name="s",
                                num_cores=NC, num_subcores=NS),
    compiler_params=pltpu.CompilerParams(
        needs_layout_passes=False, use_tc_tiling_on_sc=True),
    scratch_types=[
        pltpu.VMEM((4, 128), jnp.int32),
        pltpu.VMEM((PER_W, 128), jnp.float32),
        pltpu.VMEM((PER_W, 128), jnp.float32),
        pltpu.VMEM((PER_W, 128), jnp.float32),
        pltpu.VMEM((2 * PER_W,), jnp.float32),
        pltpu.VMEM((LANES,), jnp.float32),
        pltpu.SemaphoreType.DMA,
    ],
)


def _split_idx(x):
    pos = x[:HALF].reshape(NW, 2, 128)
    neg = x[HALF:].reshape(NW, 2, 128)
    return jnp.concatenate([pos, neg], axis=1)


@jax.jit
def kernel(batch_h, batch_t, batch_r, batch_y, ent_emb, rel_emb):
    del batch_y
    g = _stage1(batch_h, batch_t, ent_emb.T)
    rel_pad = jnp.pad(rel_emb, ((0, 0), (0, 128 - DIM)))
    pos, neg, loss_parts = _stage2(g, _split_idx(batch_r), rel_pad)
    return (jnp.sum(loss_parts), pos, neg)


# speedup vs baseline: 2.7666x; 1.0883x over previous
import jax
import jax.numpy as jnp
from jax import lax
from jax.experimental import pallas as pl
from jax.experimental.pallas import tpu as pltpu
from jax.experimental.pallas import tpu_sc as plsc

NC = 2
NS = 16
NW = NC * NS
LANES = 16

BATCH = 16384
HALF = BATCH // 2
PER_W = HALF // NW
DIM = 64
TOTAL_ENT = 1000000
RANGE = TOTAL_ENT // NW
NTC = 246
NSUB = 16
SUBCAP = 256
MYCAP = 2048
GROWS = 2 * BATCH
GDUMP = 32
RB = 8
MARGIN_F = 1.0


def _iota16():
    return lax.iota(jnp.int32, LANES)


def _stage1_body(h_ids_hbm, t_ids_hbm, ent_t, g_out,
                 hids_v, tids_v, my_ids, my_slots, subids, subslots,
                 wk_ids, wk_slots, bufA, bufB, rb, oi, smem,
                 semA, semB, semS):
    w = lax.axis_index("s") * NC + lax.axis_index("c")
    lo = w * RANGE
    hi = lo + RANGE
    base_tc = lax.div(lo, 128)
    end_tc = lax.div(hi - 1, 128)
    ntc = end_tc - base_tc + 1
    dump_row = GROWS + w % GDUMP

    pltpu.sync_copy(h_ids_hbm, hids_v)
    pltpu.sync_copy(t_ids_hbm, tids_v)

    def scan(src_v, slot_off):
        def body(v, cnt):
            ids = src_v[pl.ds(v * LANES, LANES)]
            m = (ids >= lo) & (ids < hi)
            plsc.store_compressed(my_ids.at[pl.ds(cnt, LANES)], ids, mask=m)
            sl = slot_off + v * LANES + _iota16()
            plsc.store_compressed(my_slots.at[pl.ds(cnt, LANES)], sl, mask=m)
            return cnt + plsc.all_reduce_population_count(m)[0]
        return body

    mcnt = lax.fori_loop(0, BATCH // LANES, scan(hids_v, 0), 0)
    mcnt = lax.fori_loop(0, BATCH // LANES, scan(tids_v, BATCH), mcnt)

    def bucket(v, cnts):
        ids = my_ids[pl.ds(v * LANES, LANES)]
        sls = my_slots[pl.ds(v * LANES, LANES)]
        valid = (v * LANES + _iota16()) < mcnt
        sub = lax.shift_right_logical(
            lax.shift_right_logical(ids, jnp.full((LANES,), 7, jnp.int32))
            - base_tc, jnp.full((LANES,), 4, jnp.int32))
        new = []
        for j in range(NSUB):
            mj = valid & (sub == j)
            cj = cnts[j]
            plsc.store_compressed(
                subids.at[pl.ds(j * SUBCAP + cj, LANES)], ids, mask=mj)
            plsc.store_compressed(
                subslots.at[pl.ds(j * SUBCAP + cj, LANES)], sls, mask=mj)
            new.append(cj + plsc.all_reduce_population_count(mj)[0])
        return tuple(new)

    subcnt = lax.fori_loop(0, (mcnt + LANES - 1) // LANES, bucket,
                           (0,) * NSUB)
    for j in range(NSUB):
        smem[j] = subcnt[j]

    def fire(f, buf, sem):
        tc0 = jnp.minimum(base_tc + 4 * f, (TOTAL_ENT // 128) - 3)
        off = pl.multiple_of(tc0 * 128, 128)
        pltpu.async_copy(ent_t.at[:, pl.ds(off, 512)], buf, sem)

    def drain_fetch(buf, sem):
        pltpu.make_async_copy(ent_t.at[:, pl.ds(0, 512)], buf, sem).wait()

    def drain_scatter():
        pltpu.make_async_copy(rb.at[0], g_out.at[oi.at[0]], semS).wait()

    def init_oi(b):
        plsc.store_scatter(oi.at[b], [_iota16()],
                           jnp.full((LANES,), dump_row, jnp.int32))

    for b in range(RB):
        init_oi(b)

    def process(buf, c, col_base, state):
        tc = base_tc + c
        sub = lax.div(c, NSUB)
        scnt = smem[sub]

        def svreg(v, st):
            sids = subids[pl.ds(sub * SUBCAP + v * LANES, LANES)]
            ssl = subslots[pl.ds(sub * SUBCAP + v * LANES, LANES)]
            valid = (v * LANES + _iota16()) < scnt
            m = valid & (lax.shift_right_logical(
                sids, jnp.full((LANES,), 7, jnp.int32)) == tc)
            n = plsc.all_reduce_population_count(m)[0]
            plsc.store_compressed(wk_ids.at[pl.ds(0, LANES)], sids, mask=m)
            plsc.store_compressed(wk_slots.at[pl.ds(0, LANES)], ssl, mask=m)

            def ext(e, st2):
                cur, bb, fired, drained = st2
                idv = wk_ids[pl.ds(e, LANES)]
                slv = wk_slots[pl.ds(e, LANES)]
                col = jnp.full((LANES,), (idv[0] & 127) + col_base,
                               jnp.int32)
                rows = _iota16()
                for k in range(DIM // LANES):
                    vk = plsc.load_gather(buf, [rows + k * LANES, col])
                    rb[bb, cur, pl.ds(k * LANES, LANES)] = vk
                plsc.store_scatter(
                    oi.at[bb], [jnp.full((LANES,), cur, jnp.int32)],
                    jnp.full((LANES,), slv[0], jnp.int32),
                    mask=(_iota16() == 0))
                full = cur == LANES - 1

                @pl.when(full)
                def _():
                    pltpu.async_copy(rb.at[bb], g_out.at[oi.at[bb]], semS)

                @pl.when(full & (fired >= RB - 1))
                def _():
                    drain_scatter()

                bb2 = jnp.where(full, lax.rem(bb + 1, RB), bb)

                @pl.when(full)
                def _():
                    init_oi(bb2)

                return (jnp.where(full, 0, cur + 1),
                        bb2,
                        jnp.where(full, fired + 1, fired),
                        jnp.where(full & (fired >= RB - 1),
                                  drained + 1, drained))

            return lax.fori_loop(0, n, ext, st)

        nv = lax.div(scnt + LANES - 1, LANES)
        return lax.fori_loop(0, nv, svreg, state)

    fire(0, bufA, semA)
    state = (0, 0, 0, 0)

    def proc2(buf, f, st):
        tc0f = jnp.minimum(base_tc + 4 * f, (TOTAL_ENT // 128) - 3)
        cb0 = (base_tc + 4 * f - tc0f) * 128
        st = process(buf, 4 * f, cb0, st)
        for k in (1, 2, 3):
            def kth(s2, k=k):
                return process(buf, 4 * f + k, cb0 + k * 128, s2)
            st = lax.cond(4 * f + k < ntc, kth, lambda s2: s2, st)
        return st

    def pair(i, state):
        f0 = 2 * i
        f1 = f0 + 1

        @pl.when(4 * f1 < ntc)
        def _():
            fire(f1, bufB, semB)

        drain_fetch(bufA, semA)
        state = proc2(bufA, f0, state)

        @pl.when(4 * (f0 + 2) < ntc)
        def _():
            fire(f0 + 2, bufA, semA)

        def do_b(st):
            drain_fetch(bufB, semB)
            return proc2(bufB, f1, st)

        return lax.cond(4 * f1 < ntc, do_b, lambda st: st, state)

    state = lax.fori_loop(0, (NTC + 7) // 8, pair, state)
    cur, bb, fired, drained = state

    @pl.when(cur > 0)
    def _():
        pltpu.async_copy(rb.at[bb], g_out.at[oi.at[bb]], semS)

    fired = jnp.where(cur > 0, fired + 1, fired)

    def fin(i, d):
        drain_scatter()
        return d + 1

    lax.fori_loop(0, fired - drained, fin, drained)


_stage1 = pl.kernel(
    _stage1_body,
    out_type=jax.ShapeDtypeStruct((GROWS + GDUMP, 128), jnp.float32),
    mesh=plsc.VectorSubcoreMesh(core_axis_name="c", subcore_axis_name="s",
                                num_cores=NC, num_subcores=NS),
    compiler_params=pltpu.CompilerParams(
        needs_layout_passes=False, use_tc_tiling_on_sc=True),
    scratch_types=[
        pltpu.VMEM((BATCH,), jnp.int32),
        pltpu.VMEM((BATCH,), jnp.int32),
        pltpu.VMEM((MYCAP,), jnp.int32),
        pltpu.VMEM((MYCAP,), jnp.int32),
        pltpu.VMEM((NSUB * SUBCAP,), jnp.int32),
        pltpu.VMEM((NSUB * SUBCAP,), jnp.int32),
        pltpu.VMEM((32,), jnp.int32),
        pltpu.VMEM((32,), jnp.int32),
        pltpu.VMEM((DIM, 512), jnp.float32),
        pltpu.VMEM((DIM, 512), jnp.float32),
        pltpu.VMEM((RB, LANES, 128), jnp.float32),
        pltpu.VMEM((RB, LANES), jnp.int32),
        pltpu.SMEM((NSUB,), jnp.int32),
        pltpu.SemaphoreType.DMA,
        pltpu.SemaphoreType.DMA,
        pltpu.SemaphoreType.DMA,
    ],
)


def _rsqrt_nr(s):
    i = plsc.bitcast(s, jnp.int32)
    i = jnp.full((LANES,), 0x5F3759DF, jnp.int32) - lax.shift_right_logical(
        i, jnp.full((LANES,), 1, jnp.int32))
    y = plsc.bitcast(i, jnp.float32)
    half_s = 0.5 * s
    for _ in range(3):
        y = y * (1.5 - half_s * y * y)
    return y


def _stage2_body(g_in, r_idx_hbm, rel_pad, pos_out, neg_out, loss_out,
                 r_idx_v, hbuf, tbuf, rbuf, scores_v, loss_v, sem):
    w = lax.axis_index("s") * NC + lax.axis_index("c")
    pltpu.sync_copy(r_idx_hbm.at[w], r_idx_v)

    for half in range(2):
        slot0 = half * HALF + w * PER_W
        cps = [
            pltpu.async_copy(g_in.at[pl.ds(slot0, PER_W)], hbuf, sem),
            pltpu.async_copy(g_in.at[pl.ds(BATCH + slot0, PER_W)], tbuf, sem),
            pltpu.async_copy(rel_pad.at[r_idx_v.at[2 * half]],
                             rbuf.at[pl.ds(0, 128)], sem),
            pltpu.async_copy(rel_pad.at[r_idx_v.at[2 * half + 1]],
                             rbuf.at[pl.ds(128, 128)], sem),
        ]
        for c in cps:
            c.wait()

        def group(g, carry):
            rows = g * LANES + _iota16()
            sh = jnp.zeros((LANES,), jnp.float32)
            st = jnp.zeros((LANES,), jnp.float32)
            for j in range(DIM):
                cj = jnp.full((LANES,), j, jnp.int32)
                hv = plsc.load_gather(hbuf, [rows, cj])
                tv = plsc.load_gather(tbuf, [rows, cj])
                sh = sh + hv * hv
                st = st + tv * tv
            one = jnp.full((LANES,), 1.0, jnp.float32)
            sc_h = jnp.where(sh > one, _rsqrt_nr(sh), one)
            sc_t = jnp.where(st > one, _rsqrt_nr(st), one)
            acc = jnp.zeros((LANES,), jnp.float32)
            for j in range(DIM):
                cj = jnp.full((LANES,), j, jnp.int32)
                hv = plsc.load_gather(hbuf, [rows, cj])
                tv = plsc.load_gather(tbuf, [rows, cj])
                rv = plsc.load_gather(rbuf, [rows, cj])
                acc = acc + jnp.abs(hv * sc_h + rv - tv * sc_t)
            scores_v[pl.ds(half * PER_W + g * LANES, LANES)] = acc
            return carry

        lax.fori_loop(0, PER_W // LANES, group, 0)

    def pair(m, lacc):
        pv = scores_v[pl.ds(m * LANES, LANES)]
        nv = scores_v[pl.ds(PER_W + m * LANES, LANES)]
        return lacc + jnp.maximum(pv - nv + MARGIN_F, 0.0)

    lacc = lax.fori_loop(0, PER_W // LANES, pair,
                         jnp.zeros((LANES,), jnp.float32))
    loss_v[...] = lacc

    pltpu.sync_copy(scores_v.at[pl.ds(0, PER_W)],
                    pos_out.at[pl.ds(w * PER_W, PER_W)])
    pltpu.sync_copy(scores_v.at[pl.ds(PER_W, PER_W)],
                    neg_out.at[pl.ds(w * PER_W, PER_W)])
    pltpu.sync_copy(loss_v, loss_out.at[w])


_stage2 = pl.kernel(
    _stage2_body,
    out_type=(
        jax.ShapeDtypeStruct((HALF,), jnp.float32),
        jax.ShapeDtypeStruct((HALF,), jnp.float32),
        jax.ShapeDtypeStruct((NW, LANES), jnp.float32),
    ),
    mesh=plsc.VectorSubcoreMesh(core_axis_name="c", subcore_axis_name="s",
                                num_cores=NC, num_subcores=NS),
    compiler_params=pltpu.CompilerParams(
        needs_layout_passes=False, use_tc_tiling_on_sc=True),
    scratch_types=[
        pltpu.VMEM((4, 128), jnp.int32),
        pltpu.VMEM((PER_W, 128), jnp.float32),
        pltpu.VMEM((PER_W, 128), jnp.float32),
        pltpu.VMEM((PER_W, 128), jnp.float32),
        pltpu.VMEM((2 * PER_W,), jnp.float32),
        pltpu.VMEM((LANES,), jnp.float32),
        pltpu.SemaphoreType.DMA,
    ],
)


def _split_idx(x):
    pos = x[:HALF].reshape(NW, 2, 128)
    neg = x[HALF:].reshape(NW, 2, 128)
    return jnp.concatenate([pos, neg], axis=1)


@jax.jit
def kernel(batch_h, batch_t, batch_r, batch_y, ent_emb, rel_emb):
    del batch_y
    g = _stage1(batch_h, batch_t, ent_emb.T)
    rel_pad = jnp.pad(rel_emb, ((0, 0), (0, 128 - DIM)))
    pos, neg, loss_parts = _stage2(g, _split_idx(batch_r), rel_pad)
    return (jnp.sum(loss_parts), pos, neg)


# speedup vs baseline: 2.7814x; 1.0054x over previous
import jax
import jax.numpy as jnp
from jax import lax
from jax.experimental import pallas as pl
from jax.experimental.pallas import tpu as pltpu
from jax.experimental.pallas import tpu_sc as plsc

NC = 2
NS = 16
NW = NC * NS
LANES = 16

BATCH = 16384
HALF = BATCH // 2
PER_W = HALF // NW
DIM = 64
TOTAL_ENT = 1000000
RANGE = TOTAL_ENT // NW
NTC = 246
NSUB = 16
SUBCAP = 256
MYCAP = 2048
GROWS = 2 * BATCH
GDUMP = 32
RB = 8
MARGIN_F = 1.0


def _iota16():
    return lax.iota(jnp.int32, LANES)


def _stage1_body(h_ids_hbm, t_ids_hbm, ent_t, g_out,
                 hids_v, tids_v, my_ids, my_slots, subids, subslots,
                 wk_ids, wk_slots, bufA, bufB, rb, oi, smem,
                 semA, semB, semS):
    w = lax.axis_index("s") * NC + lax.axis_index("c")
    lo = w * RANGE
    hi = lo + RANGE
    base_tc = lax.div(lo, 128)
    end_tc = lax.div(hi - 1, 128)
    ntc = end_tc - base_tc + 1
    dump_row = GROWS + w % GDUMP

    pltpu.sync_copy(h_ids_hbm, hids_v)
    pltpu.sync_copy(t_ids_hbm, tids_v)

    def scan(src_v, slot_off):
        def body(v, cnt):
            ids = src_v[pl.ds(v * LANES, LANES)]
            m = (ids >= lo) & (ids < hi)
            plsc.store_compressed(my_ids.at[pl.ds(cnt, LANES)], ids, mask=m)
            sl = slot_off + v * LANES + _iota16()
            plsc.store_compressed(my_slots.at[pl.ds(cnt, LANES)], sl, mask=m)
            return cnt + plsc.all_reduce_population_count(m)[0]
        return body

    mcnt = lax.fori_loop(0, BATCH // LANES, scan(hids_v, 0), 0)
    mcnt = lax.fori_loop(0, BATCH // LANES, scan(tids_v, BATCH), mcnt)

    def bucket(v, cnts):
        ids = my_ids[pl.ds(v * LANES, LANES)]
        sls = my_slots[pl.ds(v * LANES, LANES)]
        valid = (v * LANES + _iota16()) < mcnt
        sub = lax.shift_right_logical(
            lax.shift_right_logical(ids, jnp.full((LANES,), 7, jnp.int32))
            - base_tc, jnp.full((LANES,), 4, jnp.int32))
        new = []
        for j in range(NSUB):
            mj = valid & (sub == j)
            cj = cnts[j]
            plsc.store_compressed(
                subids.at[pl.ds(j * SUBCAP + cj, LANES)], ids, mask=mj)
            plsc.store_compressed(
                subslots.at[pl.ds(j * SUBCAP + cj, LANES)], sls, mask=mj)
            new.append(cj + plsc.all_reduce_population_count(mj)[0])
        return tuple(new)

    subcnt = lax.fori_loop(0, (mcnt + LANES - 1) // LANES, bucket,
                           (0,) * NSUB)
    for j in range(NSUB):
        smem[j] = subcnt[j]

    def fire(f, buf, sem):
        tc0 = jnp.minimum(base_tc + 4 * f, (TOTAL_ENT // 128) - 3)
        off = pl.multiple_of(tc0 * 128, 128)
        pltpu.async_copy(ent_t.at[:, pl.ds(off, 512)], buf, sem)

    def drain_fetch(buf, sem):
        pltpu.make_async_copy(ent_t.at[:, pl.ds(0, 512)], buf, sem).wait()

    def drain_scatter():
        pltpu.make_async_copy(rb.at[0], g_out.at[oi.at[0]], semS).wait()

    def init_oi(b):
        plsc.store_scatter(oi.at[b], [_iota16()],
                           jnp.full((LANES,), dump_row, jnp.int32))

    for b in range(RB):
        init_oi(b)

    def process(buf, c, col_base, state):
        tc = base_tc + c
        sub = lax.div(c, NSUB)
        scnt = smem[sub]

        def svreg(v, st):
            sids = subids[pl.ds(sub * SUBCAP + v * LANES, LANES)]
            ssl = subslots[pl.ds(sub * SUBCAP + v * LANES, LANES)]
            valid = (v * LANES + _iota16()) < scnt
            m = valid & (lax.shift_right_logical(
                sids, jnp.full((LANES,), 7, jnp.int32)) == tc)
            n = plsc.all_reduce_population_count(m)[0]
            plsc.store_compressed(wk_ids.at[pl.ds(0, LANES)], sids, mask=m)
            plsc.store_compressed(wk_slots.at[pl.ds(0, LANES)], ssl, mask=m)

            def ext(e, st2):
                cur, bb, fired, drained = st2
                idv = wk_ids[pl.ds(e, LANES)]
                slv = wk_slots[pl.ds(e, LANES)]
                col = jnp.full((LANES,), (idv[0] & 127) + col_base,
                               jnp.int32)
                rows = _iota16()
                sq = jnp.zeros((LANES,), jnp.float32)
                for k in range(DIM // LANES):
                    vk = plsc.load_gather(buf, [rows + k * LANES, col])
                    rb[bb, cur, pl.ds(k * LANES, LANES)] = vk
                    sq = sq + vk * vk
                plsc.store_scatter(
                    rb.at[bb, cur], [jnp.full((LANES,), DIM, jnp.int32)],
                    jnp.full((LANES,), jnp.sum(sq), jnp.float32),
                    mask=(_iota16() == 0))
                plsc.store_scatter(
                    oi.at[bb], [jnp.full((LANES,), cur, jnp.int32)],
                    jnp.full((LANES,), slv[0], jnp.int32),
                    mask=(_iota16() == 0))
                full = cur == LANES - 1

                @pl.when(full)
                def _():
                    pltpu.async_copy(rb.at[bb], g_out.at[oi.at[bb]], semS)

                @pl.when(full & (fired >= RB - 1))
                def _():
                    drain_scatter()

                bb2 = jnp.where(full, lax.rem(bb + 1, RB), bb)

                @pl.when(full)
                def _():
                    init_oi(bb2)

                return (jnp.where(full, 0, cur + 1),
                        bb2,
                        jnp.where(full, fired + 1, fired),
                        jnp.where(full & (fired >= RB - 1),
                                  drained + 1, drained))

            return lax.fori_loop(0, n, ext, st)

        nv = lax.div(scnt + LANES - 1, LANES)
        return lax.fori_loop(0, nv, svreg, state)

    fire(0, bufA, semA)
    state = (0, 0, 0, 0)

    def proc2(buf, f, st):
        tc0f = jnp.minimum(base_tc + 4 * f, (TOTAL_ENT // 128) - 3)
        cb0 = (base_tc + 4 * f - tc0f) * 128
        st = process(buf, 4 * f, cb0, st)
        for k in (1, 2, 3):
            def kth(s2, k=k):
                return process(buf, 4 * f + k, cb0 + k * 128, s2)
            st = lax.cond(4 * f + k < ntc, kth, lambda s2: s2, st)
        return st

    def pair(i, state):
        f0 = 2 * i
        f1 = f0 + 1

        @pl.when(4 * f1 < ntc)
        def _():
            fire(f1, bufB, semB)

        drain_fetch(bufA, semA)
        state = proc2(bufA, f0, state)

        @pl.when(4 * (f0 + 2) < ntc)
        def _():
            fire(f0 + 2, bufA, semA)

        def do_b(st):
            drain_fetch(bufB, semB)
            return proc2(bufB, f1, st)

        return lax.cond(4 * f1 < ntc, do_b, lambda st: st, state)

    state = lax.fori_loop(0, (NTC + 7) // 8, pair, state)
    cur, bb, fired, drained = state

    @pl.when(cur > 0)
    def _():
        pltpu.async_copy(rb.at[bb], g_out.at[oi.at[bb]], semS)

    fired = jnp.where(cur > 0, fired + 1, fired)

    def fin(i, d):
        drain_scatter()
        return d + 1

    lax.fori_loop(0, fired - drained, fin, drained)


_stage1 = pl.kernel(
    _stage1_body,
    out_type=jax.ShapeDtypeStruct((GROWS + GDUMP, 128), jnp.float32),
    mesh=plsc.VectorSubcoreMesh(core_axis_name="c", subcore_axis_name="s",
                                num_cores=NC, num_subcores=NS),
    compiler_params=pltpu.CompilerParams(
        needs_layout_passes=False, use_tc_tiling_on_sc=True),
    scratch_types=[
        pltpu.VMEM((BATCH,), jnp.int32),
        pltpu.VMEM((BATCH,), jnp.int32),
        pltpu.VMEM((MYCAP,), jnp.int32),
        pltpu.VMEM((MYCAP,), jnp.int32),
        pltpu.VMEM((NSUB * SUBCAP,), jnp.int32),
        pltpu.VMEM((NSUB * SUBCAP,), jnp.int32),
        pltpu.VMEM((32,), jnp.int32),
        pltpu.VMEM((32,), jnp.int32),
        pltpu.VMEM((DIM, 512), jnp.float32),
        pltpu.VMEM((DIM, 512), jnp.float32),
        pltpu.VMEM((RB, LANES, 128), jnp.float32),
        pltpu.VMEM((RB, LANES), jnp.int32),
        pltpu.SMEM((NSUB,), jnp.int32),
        pltpu.SemaphoreType.DMA,
        pltpu.SemaphoreType.DMA,
        pltpu.SemaphoreType.DMA,
    ],
)


def _rsqrt_nr(s):
    i = plsc.bitcast(s, jnp.int32)
    i = jnp.full((LANES,), 0x5F3759DF, jnp.int32) - lax.shift_right_logical(
        i, jnp.full((LANES,), 1, jnp.int32))
    y = plsc.bitcast(i, jnp.float32)
    half_s = 0.5 * s
    for _ in range(3):
        y = y * (1.5 - half_s * y * y)
    return y


def _stage2_body(g_in, r_idx_hbm, rel_pad, pos_out, neg_out, loss_out,
                 r_idx_v, hbuf, tbuf, rbuf, scores_v, loss_v, sem):
    w = lax.axis_index("s") * NC + lax.axis_index("c")
    pltpu.sync_copy(r_idx_hbm.at[w], r_idx_v)

    for half in range(2):
        slot0 = half * HALF + w * PER_W
        cps = [
            pltpu.async_copy(g_in.at[pl.ds(slot0, PER_W)], hbuf, sem),
            pltpu.async_copy(g_in.at[pl.ds(BATCH + slot0, PER_W)], tbuf, sem),
            pltpu.async_copy(rel_pad.at[r_idx_v.at[2 * half]],
                             rbuf.at[pl.ds(0, 128)], sem),
            pltpu.async_copy(rel_pad.at[r_idx_v.at[2 * half + 1]],
                             rbuf.at[pl.ds(128, 128)], sem),
        ]
        for c in cps:
            c.wait()

        def group(g, carry):
            rows = g * LANES + _iota16()
            cn = jnp.full((LANES,), DIM, jnp.int32)
            sh = plsc.load_gather(hbuf, [rows, cn])
            st = plsc.load_gather(tbuf, [rows, cn])
            one = jnp.full((LANES,), 1.0, jnp.float32)
            sc_h = jnp.where(sh > one, _rsqrt_nr(sh), one)
            sc_t = jnp.where(st > one, _rsqrt_nr(st), one)
            acc = jnp.zeros((LANES,), jnp.float32)
            for j in range(DIM):
                cj = jnp.full((LANES,), j, jnp.int32)
                hv = plsc.load_gather(hbuf, [rows, cj])
                tv = plsc.load_gather(tbuf, [rows, cj])
                rv = plsc.load_gather(rbuf, [rows, cj])
                acc = acc + jnp.abs(hv * sc_h + rv - tv * sc_t)
            scores_v[pl.ds(half * PER_W + g * LANES, LANES)] = acc
            return carry

        lax.fori_loop(0, PER_W // LANES, group, 0)

    def pair(m, lacc):
        pv = scores_v[pl.ds(m * LANES, LANES)]
        nv = scores_v[pl.ds(PER_W + m * LANES, LANES)]
        return lacc + jnp.maximum(pv - nv + MARGIN_F, 0.0)

    lacc = lax.fori_loop(0, PER_W // LANES, pair,
                         jnp.zeros((LANES,), jnp.float32))
    loss_v[...] = lacc

    pltpu.sync_copy(scores_v.at[pl.ds(0, PER_W)],
                    pos_out.at[pl.ds(w * PER_W, PER_W)])
    pltpu.sync_copy(scores_v.at[pl.ds(PER_W, PER_W)],
                    neg_out.at[pl.ds(w * PER_W, PER_W)])
    pltpu.sync_copy(loss_v, loss_out.at[w])


_stage2 = pl.kernel(
    _stage2_body,
    out_type=(
        jax.ShapeDtypeStruct((HALF,), jnp.float32),
        jax.ShapeDtypeStruct((HALF,), jnp.float32),
        jax.ShapeDtypeStruct((NW, LANES), jnp.float32),
    ),
    mesh=plsc.VectorSubcoreMesh(core_axis_name="c", subcore_axis_name="s",
                                num_cores=NC, num_subcores=NS),
    compiler_params=pltpu.CompilerParams(
        needs_layout_passes=False, use_tc_tiling_on_sc=True),
    scratch_types=[
        pltpu.VMEM((4, 128), jnp.int32),
        pltpu.VMEM((PER_W, 128), jnp.float32),
        pltpu.VMEM((PER_W, 128), jnp.float32),
        pltpu.VMEM((PER_W, 128), jnp.float32),
        pltpu.VMEM((2 * PER_W,), jnp.float32),
        pltpu.VMEM((LANES,), jnp.float32),
        pltpu.SemaphoreType.DMA,
    ],
)


def _split_idx(x):
    pos = x[:HALF].reshape(NW, 2, 128)
    neg = x[HALF:].reshape(NW, 2, 128)
    return jnp.concatenate([pos, neg], axis=1)


@jax.jit
def kernel(batch_h, batch_t, batch_r, batch_y, ent_emb, rel_emb):
    del batch_y
    g = _stage1(batch_h, batch_t, ent_emb.T)
    rel_pad = jnp.pad(rel_emb, ((0, 0), (0, 128 - DIM)))
    pos, neg, loss_parts = _stage2(g, _split_idx(batch_r), rel_pad)
    return (jnp.sum(loss_parts), pos, neg)


# speedup vs baseline: 2.8433x; 1.0223x over previous
import jax
import jax.numpy as jnp
from jax import lax
from jax.experimental import pallas as pl
from jax.experimental.pallas import tpu as pltpu
from jax.experimental.pallas import tpu_sc as plsc

NC = 2
NS = 16
NW = NC * NS
LANES = 16

BATCH = 16384
HALF = BATCH // 2
PER_W = HALF // NW
DIM = 64
TOTAL_ENT = 1000000
RANGE = TOTAL_ENT // NW
NTC = 246
NSUB = 16
SUBCAP = 256
MYCAP = 2048
GROWS = 2 * BATCH
GDUMP = 32
RB = 8
MARGIN_F = 1.0


def _iota16():
    return lax.iota(jnp.int32, LANES)


def _stage1_body(h_ids_hbm, t_ids_hbm, ent_t, g_out,
                 hids_v, tids_v, my_ids, my_slots, subids, subslots,
                 wk_ids, wk_slots, bufA, bufB, rb, oi, smem,
                 semA, semB, semS):
    w = lax.axis_index("s") * NC + lax.axis_index("c")
    lo = w * RANGE
    hi = lo + RANGE
    base_tc = lax.div(lo, 128)
    end_tc = lax.div(hi - 1, 128)
    ntc = end_tc - base_tc + 1
    dump_row = GROWS + w % GDUMP

    pltpu.sync_copy(h_ids_hbm, hids_v)
    pltpu.sync_copy(t_ids_hbm, tids_v)

    def scan(src_v, slot_off):
        def body(v, cnt):
            ids = src_v[pl.ds(v * LANES, LANES)]
            m = (ids >= lo) & (ids < hi)
            plsc.store_compressed(my_ids.at[pl.ds(cnt, LANES)], ids, mask=m)
            sl = slot_off + v * LANES + _iota16()
            plsc.store_compressed(my_slots.at[pl.ds(cnt, LANES)], sl, mask=m)
            return cnt + plsc.all_reduce_population_count(m)[0]
        return body

    mcnt = lax.fori_loop(0, BATCH // LANES, scan(hids_v, 0), 0)
    mcnt = lax.fori_loop(0, BATCH // LANES, scan(tids_v, BATCH), mcnt)

    def bucket(v, cnts):
        ids = my_ids[pl.ds(v * LANES, LANES)]
        sls = my_slots[pl.ds(v * LANES, LANES)]
        valid = (v * LANES + _iota16()) < mcnt
        sub = lax.shift_right_logical(
            lax.shift_right_logical(ids, jnp.full((LANES,), 7, jnp.int32))
            - base_tc, jnp.full((LANES,), 4, jnp.int32))
        new = []
        for j in range(NSUB):
            mj = valid & (sub == j)
            cj = cnts[j]
            plsc.store_compressed(
                subids.at[pl.ds(j * SUBCAP + cj, LANES)], ids, mask=mj)
            plsc.store_compressed(
                subslots.at[pl.ds(j * SUBCAP + cj, LANES)], sls, mask=mj)
            new.append(cj + plsc.all_reduce_population_count(mj)[0])
        return tuple(new)

    subcnt = lax.fori_loop(0, (mcnt + LANES - 1) // LANES, bucket,
                           (0,) * NSUB)
    for j in range(NSUB):
        smem[j] = subcnt[j]

    def fire(f, buf, sem):
        tc0 = jnp.minimum(base_tc + 4 * f, (TOTAL_ENT // 128) - 3)
        off = pl.multiple_of(tc0 * 128, 128)
        pltpu.async_copy(ent_t.at[:, pl.ds(off, 512)], buf, sem)

    def drain_fetch(buf, sem):
        pltpu.make_async_copy(ent_t.at[:, pl.ds(0, 512)], buf, sem).wait()

    def drain_scatter():
        pltpu.make_async_copy(rb.at[0], g_out.at[oi.at[0]], semS).wait()

    def init_oi(b):
        plsc.store_scatter(oi.at[b], [_iota16()],
                           jnp.full((LANES,), dump_row, jnp.int32))

    for b in range(RB):
        init_oi(b)

    def process(buf, c, col_base, state):
        tc = base_tc + c
        sub = lax.div(c, NSUB)
        scnt = smem[sub]

        def svreg(v, st):
            sids = subids[pl.ds(sub * SUBCAP + v * LANES, LANES)]
            ssl = subslots[pl.ds(sub * SUBCAP + v * LANES, LANES)]
            valid = (v * LANES + _iota16()) < scnt
            m = valid & (lax.shift_right_logical(
                sids, jnp.full((LANES,), 7, jnp.int32)) == tc)
            n = plsc.all_reduce_population_count(m)[0]
            plsc.store_compressed(wk_ids.at[pl.ds(0, LANES)], sids, mask=m)
            plsc.store_compressed(wk_slots.at[pl.ds(0, LANES)], ssl, mask=m)

            def ext(e, st2):
                cur, bb, fired, drained = st2
                idv = wk_ids[pl.ds(e, LANES)]
                slv = wk_slots[pl.ds(e, LANES)]
                col = jnp.full((LANES,), (idv[0] & 127) + col_base,
                               jnp.int32)
                rows = _iota16()
                sq = jnp.zeros((LANES,), jnp.float32)
                for k in range(DIM // LANES):
                    vk = plsc.load_gather(buf, [rows + k * LANES, col])
                    rb[bb, cur, pl.ds(k * LANES, LANES)] = vk
                    sq = sq + vk * vk
                plsc.store_scatter(
                    rb.at[bb, cur], [jnp.full((LANES,), DIM, jnp.int32)],
                    jnp.full((LANES,), jnp.sum(sq), jnp.float32),
                    mask=(_iota16() == 0))
                plsc.store_scatter(
                    oi.at[bb], [jnp.full((LANES,), cur, jnp.int32)],
                    jnp.full((LANES,), slv[0], jnp.int32),
                    mask=(_iota16() == 0))
                full = cur == LANES - 1

                @pl.when(full)
                def _():
                    pltpu.async_copy(rb.at[bb], g_out.at[oi.at[bb]], semS)

                @pl.when(full & (fired >= RB - 1))
                def _():
                    drain_scatter()

                bb2 = jnp.where(full, lax.rem(bb + 1, RB), bb)

                @pl.when(full)
                def _():
                    init_oi(bb2)

                return (jnp.where(full, 0, cur + 1),
                        bb2,
                        jnp.where(full, fired + 1, fired),
                        jnp.where(full & (fired >= RB - 1),
                                  drained + 1, drained))

            return lax.fori_loop(0, n, ext, st)

        nv = lax.div(scnt + LANES - 1, LANES)
        return lax.fori_loop(0, nv, svreg, state)

    fire(0, bufA, semA)
    state = (0, 0, 0, 0)

    def proc2(buf, f, st):
        tc0f = jnp.minimum(base_tc + 4 * f, (TOTAL_ENT // 128) - 3)
        cb0 = (base_tc + 4 * f - tc0f) * 128
        st = process(buf, 4 * f, cb0, st)
        for k in (1, 2, 3):
            def kth(s2, k=k):
                return process(buf, 4 * f + k, cb0 + k * 128, s2)
            st = lax.cond(4 * f + k < ntc, kth, lambda s2: s2, st)
        return st

    def pair(i, state):
        f0 = 2 * i
        f1 = f0 + 1

        @pl.when(4 * f1 < ntc)
        def _():
            fire(f1, bufB, semB)

        drain_fetch(bufA, semA)
        state = proc2(bufA, f0, state)

        @pl.when(4 * (f0 + 2) < ntc)
        def _():
            fire(f0 + 2, bufA, semA)

        def do_b(st):
            drain_fetch(bufB, semB)
            return proc2(bufB, f1, st)

        return lax.cond(4 * f1 < ntc, do_b, lambda st: st, state)

    state = lax.fori_loop(0, (NTC + 7) // 8, pair, state)
    cur, bb, fired, drained = state

    @pl.when(cur > 0)
    def _():
        pltpu.async_copy(rb.at[bb], g_out.at[oi.at[bb]], semS)

    fired = jnp.where(cur > 0, fired + 1, fired)

    def fin(i, d):
        drain_scatter()
        return d + 1

    lax.fori_loop(0, fired - drained, fin, drained)


_stage1 = pl.kernel(
    _stage1_body,
    out_type=jax.ShapeDtypeStruct((GROWS + GDUMP, 128), jnp.float32),
    mesh=plsc.VectorSubcoreMesh(core_axis_name="c", subcore_axis_name="s",
                                num_cores=NC, num_subcores=NS),
    compiler_params=pltpu.CompilerParams(
        needs_layout_passes=False, use_tc_tiling_on_sc=True),
    scratch_types=[
        pltpu.VMEM((BATCH,), jnp.int32),
        pltpu.VMEM((BATCH,), jnp.int32),
        pltpu.VMEM((MYCAP,), jnp.int32),
        pltpu.VMEM((MYCAP,), jnp.int32),
        pltpu.VMEM((NSUB * SUBCAP,), jnp.int32),
        pltpu.VMEM((NSUB * SUBCAP,), jnp.int32),
        pltpu.VMEM((32,), jnp.int32),
        pltpu.VMEM((32,), jnp.int32),
        pltpu.VMEM((DIM, 512), jnp.float32),
        pltpu.VMEM((DIM, 512), jnp.float32),
        pltpu.VMEM((RB, LANES, 128), jnp.float32),
        pltpu.VMEM((RB, LANES), jnp.int32),
        pltpu.SMEM((NSUB,), jnp.int32),
        pltpu.SemaphoreType.DMA,
        pltpu.SemaphoreType.DMA,
        pltpu.SemaphoreType.DMA,
    ],
)


def _rsqrt_nr(s):
    i = plsc.bitcast(s, jnp.int32)
    i = jnp.full((LANES,), 0x5F3759DF, jnp.int32) - lax.shift_right_logical(
        i, jnp.full((LANES,), 1, jnp.int32))
    y = plsc.bitcast(i, jnp.float32)
    half_s = 0.5 * s
    for _ in range(3):
        y = y * (1.5 - half_s * y * y)
    return y


def _stage2_body(g_in, r_idx_hbm, rel_pad, pos_out, neg_out, loss_out,
                 r_idx_v, hbuf, tbuf, rbuf, scores_v, loss_v, sem):
    w = lax.axis_index("s") * NC + lax.axis_index("c")
    pltpu.sync_copy(r_idx_hbm.at[w], r_idx_v)

    def fire(c):
        p = c % 2
        slot0 = (c // 2) * HALF + w * PER_W + (c % 2) * 128
        return [
            pltpu.async_copy(g_in.at[pl.ds(slot0, 128)], hbuf.at[p], sem),
            pltpu.async_copy(g_in.at[pl.ds(BATCH + slot0, 128)],
                             tbuf.at[p], sem),
            pltpu.async_copy(rel_pad.at[r_idx_v.at[c]], rbuf.at[p], sem),
        ]

    pend = fire(0)
    for c in range(4):
        nxt = fire(c + 1) if c < 3 else []
        for d in pend:
            d.wait()
        pend = nxt
        p = c % 2

        def group(g, carry):
            rows = g * LANES + _iota16()
            cn = jnp.full((LANES,), DIM, jnp.int32)
            sh = plsc.load_gather(hbuf.at[p], [rows, cn])
            st = plsc.load_gather(tbuf.at[p], [rows, cn])
            one = jnp.full((LANES,), 1.0, jnp.float32)
            sc_h = jnp.where(sh > one, _rsqrt_nr(sh), one)
            sc_t = jnp.where(st > one, _rsqrt_nr(st), one)
            acc = jnp.zeros((LANES,), jnp.float32)
            for j in range(DIM):
                cj = jnp.full((LANES,), j, jnp.int32)
                hv = plsc.load_gather(hbuf.at[p], [rows, cj])
                tv = plsc.load_gather(tbuf.at[p], [rows, cj])
                rv = plsc.load_gather(rbuf.at[p], [rows, cj])
                acc = acc + jnp.abs(hv * sc_h + rv - tv * sc_t)
            scores_v[pl.ds(c * 128 + g * LANES, LANES)] = acc
            return carry

        lax.fori_loop(0, 128 // LANES, group, 0)

    def pair(m, lacc):
        pv = scores_v[pl.ds(m * LANES, LANES)]
        nv = scores_v[pl.ds(PER_W + m * LANES, LANES)]
        return lacc + jnp.maximum(pv - nv + MARGIN_F, 0.0)

    lacc = lax.fori_loop(0, PER_W // LANES, pair,
                         jnp.zeros((LANES,), jnp.float32))
    loss_v[...] = lacc

    pltpu.sync_copy(scores_v.at[pl.ds(0, PER_W)],
                    pos_out.at[pl.ds(w * PER_W, PER_W)])
    pltpu.sync_copy(scores_v.at[pl.ds(PER_W, PER_W)],
                    neg_out.at[pl.ds(w * PER_W, PER_W)])
    pltpu.sync_copy(loss_v, loss_out.at[w])


_stage2 = pl.kernel(
    _stage2_body,
    out_type=(
        jax.ShapeDtypeStruct((HALF,), jnp.float32),
        jax.ShapeDtypeStruct((HALF,), jnp.float32),
        jax.ShapeDtypeStruct((NW, LANES), jnp.float32),
    ),
    mesh=plsc.VectorSubcoreMesh(core_axis_name="c", subcore_axis_name="s",
                                num_cores=NC, num_subcores=NS),
    compiler_params=pltpu.CompilerParams(
        needs_layout_passes=False, use_tc_tiling_on_sc=True),
    scratch_types=[
        pltpu.VMEM((4, 128), jnp.int32),
        pltpu.VMEM((2, 128, 128), jnp.float32),
        pltpu.VMEM((2, 128, 128), jnp.float32),
        pltpu.VMEM((2, 128, 128), jnp.float32),
        pltpu.VMEM((2 * PER_W,), jnp.float32),
        pltpu.VMEM((LANES,), jnp.float32),
        pltpu.SemaphoreType.DMA,
    ],
)


def _split_idx(x):
    pos = x[:HALF].reshape(NW, 2, 128)
    neg = x[HALF:].reshape(NW, 2, 128)
    return jnp.concatenate([pos, neg], axis=1)


@jax.jit
def kernel(batch_h, batch_t, batch_r, batch_y, ent_emb, rel_emb):
    del batch_y
    g = _stage1(batch_h, batch_t, ent_emb.T)
    rel_pad = jnp.pad(rel_emb, ((0, 0), (0, 128 - DIM)))
    pos, neg, loss_parts = _stage2(g, _split_idx(batch_r), rel_pad)
    return (jnp.sum(loss_parts), pos, neg)


# speedup vs baseline: 3.2944x; 1.1586x over previous
import jax
import jax.numpy as jnp
from jax import lax
from jax.experimental import pallas as pl
from jax.experimental.pallas import tpu as pltpu
from jax.experimental.pallas import tpu_sc as plsc

NC = 2
NS = 16
NW = NC * NS
LANES = 16

BATCH = 16384
HALF = BATCH // 2
PER_W = HALF // NW
DIM = 64
TOTAL_ENT = 1000000
RANGE = TOTAL_ENT // NW
NTC = 246
NSUB = 16
SUBCAP = 256
MYCAP = 2048
GROWS = 2 * BATCH
GDUMP = 32
RB = 8
MARGIN_F = 1.0


def _iota16():
    return lax.iota(jnp.int32, LANES)


def _stage1_body(h_ids_hbm, t_ids_hbm, ent_t, g_out,
                 hids_v, tids_v, my_ids, my_slots, subids, subslots,
                 wk_ids, wk_slots, bufA, bufB, rb, oi, smem,
                 semA, semB, semS):
    w = lax.axis_index("s") * NC + lax.axis_index("c")
    lo = w * RANGE
    hi = lo + RANGE
    base_tc = lax.div(lo, 128)
    end_tc = lax.div(hi - 1, 128)
    ntc = end_tc - base_tc + 1
    dump_row = GROWS + w % GDUMP

    pltpu.sync_copy(h_ids_hbm, hids_v)
    pltpu.sync_copy(t_ids_hbm, tids_v)

    def scan(src_v, slot_off):
        def body(v, cnt):
            ids = src_v[pl.ds(v * LANES, LANES)]
            m = (ids >= lo) & (ids < hi)
            plsc.store_compressed(my_ids.at[pl.ds(cnt, LANES)], ids, mask=m)
            sl = slot_off + v * LANES + _iota16()
            plsc.store_compressed(my_slots.at[pl.ds(cnt, LANES)], sl, mask=m)
            return cnt + plsc.all_reduce_population_count(m)[0]
        return body

    mcnt = lax.fori_loop(0, BATCH // LANES, scan(hids_v, 0), 0)
    mcnt = lax.fori_loop(0, BATCH // LANES, scan(tids_v, BATCH), mcnt)

    def bucket(v, cnts):
        ids = my_ids[pl.ds(v * LANES, LANES)]
        sls = my_slots[pl.ds(v * LANES, LANES)]
        valid = (v * LANES + _iota16()) < mcnt
        sub = lax.shift_right_logical(
            lax.shift_right_logical(ids, jnp.full((LANES,), 7, jnp.int32))
            - base_tc, jnp.full((LANES,), 4, jnp.int32))
        new = []
        for j in range(NSUB):
            mj = valid & (sub == j)
            cj = cnts[j]
            plsc.store_compressed(
                subids.at[pl.ds(j * SUBCAP + cj, LANES)], ids, mask=mj)
            plsc.store_compressed(
                subslots.at[pl.ds(j * SUBCAP + cj, LANES)], sls, mask=mj)
            new.append(cj + plsc.all_reduce_population_count(mj)[0])
        return tuple(new)

    subcnt = lax.fori_loop(0, (mcnt + LANES - 1) // LANES, bucket,
                           (0,) * NSUB)
    for j in range(NSUB):
        smem[j] = subcnt[j]

    def fire(f, buf, sem):
        tc0 = jnp.minimum(base_tc + 4 * f, (TOTAL_ENT // 128) - 3)
        off = pl.multiple_of(tc0 * 128, 128)
        pltpu.async_copy(ent_t.at[:, pl.ds(off, 512)], buf, sem)

    def drain_fetch(buf, sem):
        pltpu.make_async_copy(ent_t.at[:, pl.ds(0, 512)], buf, sem).wait()

    def drain_scatter():
        pltpu.make_async_copy(rb.at[0], g_out.at[oi.at[0]], semS).wait()

    def init_oi(b):
        plsc.store_scatter(oi.at[b], [_iota16()],
                           jnp.full((LANES,), dump_row, jnp.int32))

    for b in range(RB):
        init_oi(b)

    def process(buf, c, col_base, state):
        tc = base_tc + c
        sub = lax.div(c, NSUB)
        scnt = smem[sub]

        def svreg(v, st):
            sids = subids[pl.ds(sub * SUBCAP + v * LANES, LANES)]
            ssl = subslots[pl.ds(sub * SUBCAP + v * LANES, LANES)]
            valid = (v * LANES + _iota16()) < scnt
            m = valid & (lax.shift_right_logical(
                sids, jnp.full((LANES,), 7, jnp.int32)) == tc)
            n = plsc.all_reduce_population_count(m)[0]
            plsc.store_compressed(wk_ids.at[pl.ds(0, LANES)], sids, mask=m)
            plsc.store_compressed(wk_slots.at[pl.ds(0, LANES)], ssl, mask=m)

            def ext(e, st2):
                cur, bb, fired, drained = st2
                idv = wk_ids[pl.ds(e, LANES)]
                slv = wk_slots[pl.ds(e, LANES)]
                col = jnp.full((LANES,), (idv[0] & 127) + col_base,
                               jnp.int32)
                rows = _iota16()
                sq = jnp.zeros((LANES,), jnp.float32)
                for k in range(DIM // LANES):
                    vk = plsc.load_gather(buf, [rows + k * LANES, col])
                    rb[bb, cur, pl.ds(k * LANES, LANES)] = vk
                    sq = sq + vk * vk
                plsc.store_scatter(
                    rb.at[bb, cur], [jnp.full((LANES,), DIM, jnp.int32)],
                    jnp.full((LANES,), jnp.sum(sq), jnp.float32),
                    mask=(_iota16() == 0))
                plsc.store_scatter(
                    oi.at[bb], [jnp.full((LANES,), cur, jnp.int32)],
                    jnp.full((LANES,), slv[0], jnp.int32),
                    mask=(_iota16() == 0))
                full = cur == LANES - 1

                @pl.when(full)
                def _():
                    pltpu.async_copy(rb.at[bb], g_out.at[oi.at[bb]], semS)

                @pl.when(full & (fired >= RB - 1))
                def _():
                    drain_scatter()

                bb2 = jnp.where(full, lax.rem(bb + 1, RB), bb)

                @pl.when(full)
                def _():
                    init_oi(bb2)

                return (jnp.where(full, 0, cur + 1),
                        bb2,
                        jnp.where(full, fired + 1, fired),
                        jnp.where(full & (fired >= RB - 1),
                                  drained + 1, drained))

            return lax.fori_loop(0, n, ext, st)

        nv = lax.div(scnt + LANES - 1, LANES)
        return lax.fori_loop(0, nv, svreg, state)

    fire(0, bufA, semA)
    state = (0, 0, 0, 0)

    def proc2(buf, f, st):
        tc0f = jnp.minimum(base_tc + 4 * f, (TOTAL_ENT // 128) - 3)
        cb0 = (base_tc + 4 * f - tc0f) * 128
        st = process(buf, 4 * f, cb0, st)
        for k in (1, 2, 3):
            def kth(s2, k=k):
                return process(buf, 4 * f + k, cb0 + k * 128, s2)
            st = lax.cond(4 * f + k < ntc, kth, lambda s2: s2, st)
        return st

    def pair(i, state):
        f0 = 2 * i
        f1 = f0 + 1

        @pl.when(4 * f1 < ntc)
        def _():
            fire(f1, bufB, semB)

        drain_fetch(bufA, semA)
        state = proc2(bufA, f0, state)

        @pl.when(4 * (f0 + 2) < ntc)
        def _():
            fire(f0 + 2, bufA, semA)

        def do_b(st):
            drain_fetch(bufB, semB)
            return proc2(bufB, f1, st)

        return lax.cond(4 * f1 < ntc, do_b, lambda st: st, state)

    state = lax.fori_loop(0, (NTC + 7) // 8, pair, state)
    cur, bb, fired, drained = state

    @pl.when(cur > 0)
    def _():
        pltpu.async_copy(rb.at[bb], g_out.at[oi.at[bb]], semS)

    fired = jnp.where(cur > 0, fired + 1, fired)

    def fin(i, d):
        drain_scatter()
        return d + 1

    lax.fori_loop(0, fired - drained, fin, drained)


_stage1 = pl.kernel(
    _stage1_body,
    out_type=jax.ShapeDtypeStruct((GROWS + GDUMP, 128), jnp.float32),
    mesh=plsc.VectorSubcoreMesh(core_axis_name="c", subcore_axis_name="s",
                                num_cores=NC, num_subcores=NS),
    compiler_params=pltpu.CompilerParams(
        needs_layout_passes=False, use_tc_tiling_on_sc=True),
    scratch_types=[
        pltpu.VMEM((BATCH,), jnp.int32),
        pltpu.VMEM((BATCH,), jnp.int32),
        pltpu.VMEM((MYCAP,), jnp.int32),
        pltpu.VMEM((MYCAP,), jnp.int32),
        pltpu.VMEM((NSUB * SUBCAP,), jnp.int32),
        pltpu.VMEM((NSUB * SUBCAP,), jnp.int32),
        pltpu.VMEM((32,), jnp.int32),
        pltpu.VMEM((32,), jnp.int32),
        pltpu.VMEM((DIM, 512), jnp.float32),
        pltpu.VMEM((DIM, 512), jnp.float32),
        pltpu.VMEM((RB, LANES, 128), jnp.float32),
        pltpu.VMEM((RB, LANES), jnp.int32),
        pltpu.SMEM((NSUB,), jnp.int32),
        pltpu.SemaphoreType.DMA,
        pltpu.SemaphoreType.DMA,
        pltpu.SemaphoreType.DMA,
    ],
)


def _rsqrt_nr(s):
    i = plsc.bitcast(s, jnp.int32)
    i = jnp.full((LANES,), 0x5F3759DF, jnp.int32) - lax.shift_right_logical(
        i, jnp.full((LANES,), 1, jnp.int32))
    y = plsc.bitcast(i, jnp.float32)
    half_s = 0.5 * s
    for _ in range(3):
        y = y * (1.5 - half_s * y * y)
    return y


def _stage2_body(g_in, r_idx_hbm, rel_pad, pos_out, neg_out, loss_out,
                 r_idx_v, hbuf, tbuf, rbuf, scores_v, sch_v, sct_v,
                 loss_v, sem):
    w = lax.axis_index("s") * NC + lax.axis_index("c")
    pltpu.sync_copy(r_idx_hbm.at[w], r_idx_v)

    def fire(c):
        p = c % 2
        slot0 = (c // 2) * HALF + w * PER_W + (c % 2) * 128
        return [
            pltpu.async_copy(g_in.at[pl.ds(slot0, 128)], hbuf.at[p], sem),
            pltpu.async_copy(g_in.at[pl.ds(BATCH + slot0, 128)],
                             tbuf.at[p], sem),
            pltpu.async_copy(rel_pad.at[r_idx_v.at[c]], rbuf.at[p], sem),
        ]

    pend = fire(0)
    for c in range(4):
        nxt = fire(c + 1) if c < 3 else []
        for d in pend:
            d.wait()
        pend = nxt
        p = c % 2

        def scales(g, carry):
            rows = g * LANES + _iota16()
            cn = jnp.full((LANES,), DIM, jnp.int32)
            sh = plsc.load_gather(hbuf.at[p], [rows, cn])
            st = plsc.load_gather(tbuf.at[p], [rows, cn])
            one = jnp.full((LANES,), 1.0, jnp.float32)
            sch_v[pl.ds(g * LANES, LANES)] = jnp.where(
                sh > one, _rsqrt_nr(sh), one)
            sct_v[pl.ds(g * LANES, LANES)] = jnp.where(
                st > one, _rsqrt_nr(st), one)
            return carry

        lax.fori_loop(0, 128 // LANES, scales, 0)

        def row(e, carry):
            shs = jnp.full((LANES,), sch_v[pl.ds(e, LANES)][0], jnp.float32)
            sts = jnp.full((LANES,), sct_v[pl.ds(e, LANES)][0], jnp.float32)
            parts = []
            for k in range(DIM // LANES):
                sl = pl.ds(k * LANES, LANES)
                hv = hbuf[p, e, sl]
                tv = tbuf[p, e, sl]
                rv = rbuf[p, e, sl]
                parts.append(jnp.abs(hv * shs + rv - tv * sts))
            tot = jnp.sum((parts[0] + parts[1]) + (parts[2] + parts[3]))
            plsc.store_scatter(
                scores_v, [jnp.full((LANES,), c * 128 + e, jnp.int32)],
                jnp.full((LANES,), tot, jnp.float32),
                mask=(_iota16() == 0))
            return carry

        lax.fori_loop(0, 128, row, 0)

    def pair(m, lacc):
        pv = scores_v[pl.ds(m * LANES, LANES)]
        nv = scores_v[pl.ds(PER_W + m * LANES, LANES)]
        return lacc + jnp.maximum(pv - nv + MARGIN_F, 0.0)

    lacc = lax.fori_loop(0, PER_W // LANES, pair,
                         jnp.zeros((LANES,), jnp.float32))
    loss_v[...] = lacc

    pltpu.sync_copy(scores_v.at[pl.ds(0, PER_W)],
                    pos_out.at[pl.ds(w * PER_W, PER_W)])
    pltpu.sync_copy(scores_v.at[pl.ds(PER_W, PER_W)],
                    neg_out.at[pl.ds(w * PER_W, PER_W)])
    pltpu.sync_copy(loss_v, loss_out.at[w])


_stage2 = pl.kernel(
    _stage2_body,
    out_type=(
        jax.ShapeDtypeStruct((HALF,), jnp.float32),
        jax.ShapeDtypeStruct((HALF,), jnp.float32),
        jax.ShapeDtypeStruct((NW, LANES), jnp.float32),
    ),
    mesh=plsc.VectorSubcoreMesh(core_axis_name="c", subcore_axis_name="s",
                                num_cores=NC, num_subcores=NS),
    compiler_params=pltpu.CompilerParams(
        needs_layout_passes=False, use_tc_tiling_on_sc=True),
    scratch_types=[
        pltpu.VMEM((4, 128), jnp.int32),
        pltpu.VMEM((2, 128, 128), jnp.float32),
        pltpu.VMEM((2, 128, 128), jnp.float32),
        pltpu.VMEM((2, 128, 128), jnp.float32),
        pltpu.VMEM((2 * PER_W,), jnp.float32),
        pltpu.VMEM((144,), jnp.float32),
        pltpu.VMEM((144,), jnp.float32),
        pltpu.VMEM((LANES,), jnp.float32),
        pltpu.SemaphoreType.DMA,
    ],
)


def _split_idx(x):
    pos = x[:HALF].reshape(NW, 2, 128)
    neg = x[HALF:].reshape(NW, 2, 128)
    return jnp.concatenate([pos, neg], axis=1)


@jax.jit
def kernel(batch_h, batch_t, batch_r, batch_y, ent_emb, rel_emb):
    del batch_y
    g = _stage1(batch_h, batch_t, ent_emb.T)
    rel_pad = jnp.pad(rel_emb, ((0, 0), (0, 128 - DIM)))
    pos, neg, loss_parts = _stage2(g, _split_idx(batch_r), rel_pad)
    return (jnp.sum(loss_parts), pos, neg)
